# Initial kernel scaffold; baseline (speedup 1.0000x reference)
import jax
import jax.numpy as jnp
from jax.experimental import pallas as pl


def kernel(x, edge_index, edge_attr, batch, mW1, mb1, mW2, mb2, mW3, mb3, nW1, nb1, nW2, nb2, nW3, nb3, lW, lb):
    raise NotImplementedError("write your pallas kernel here")


# speedup vs baseline: 2.7745x; 2.7745x over previous
import functools

import jax
import jax.numpy as jnp
from jax import lax
from jax.experimental import pallas as pl
from jax.experimental.pallas import tpu as pltpu
from jax.experimental.pallas import tpu_sc as plsc

N = 10000
E = 320000
NF = 128
NEF = 16
MSG = 128
HID = 300
NH = 128
NP = 2
NG = 64

NC = 2
NS = 16
NW = NC * NS
PER_W = E // NW
C = 80
NCHUNK = PER_W // C
NAGG = 10240
ROWS_PER_TILE = NAGG // NS


def _gather_body(x_hbm, src_hbm, dst_hbm, xj_hbm, xi_hbm,
                 idx_a, rows_a, idx_b, rows_b, sem_a, sem_b):
    c = lax.axis_index("c")
    s = lax.axis_index("s")
    base = (c * NS + s) * PER_W

    @pl.loop(0, NCHUNK)
    def _(j):
        off = base + j * C
        pltpu.sync_copy(src_hbm.at[pl.ds(off, C)], idx_a)
        cp_a = pltpu.async_copy(x_hbm.at[idx_a], rows_a, sem_a)
        pltpu.sync_copy(dst_hbm.at[pl.ds(off, C)], idx_b)
        cp_b = pltpu.async_copy(x_hbm.at[idx_b], rows_b, sem_b)
        cp_a.wait()
        pltpu.sync_copy(rows_a, xj_hbm.at[pl.ds(off, C)])
        cp_b.wait()
        pltpu.sync_copy(rows_b, xi_hbm.at[pl.ds(off, C)])


def _sc_gather(x, src, dst):
    mesh = plsc.VectorSubcoreMesh(core_axis_name="c", subcore_axis_name="s")
    f = pl.kernel(
        _gather_body,
        out_type=(
            jax.ShapeDtypeStruct((E, NF), jnp.float32),
            jax.ShapeDtypeStruct((E, NF), jnp.float32),
        ),
        mesh=mesh,
        scratch_types=[
            pltpu.VMEM((C,), jnp.int32),
            pltpu.VMEM((C, NF), jnp.float32),
            pltpu.VMEM((C,), jnp.int32),
            pltpu.VMEM((C, NF), jnp.float32),
            pltpu.SemaphoreType.DMA,
            pltpu.SemaphoreType.DMA,
        ],
    )
    return f(x, src, dst)


def _scatter_body(msg_hbm, dst_hbm, z_hbm, out_hbm, idx_v, rows_v, acc_sh):
    c = lax.axis_index("c")
    s = lax.axis_index("s")
    pltpu.sync_copy(z_hbm, acc_sh.at[pl.ds(s * ROWS_PER_TILE, ROWS_PER_TILE)])
    plsc.subcore_barrier()

    base = (c * NS + s) * PER_W

    @pl.loop(0, NCHUNK)
    def _(j):
        off = base + j * C
        pltpu.sync_copy(dst_hbm.at[pl.ds(off, C)], idx_v)
        pltpu.sync_copy(msg_hbm.at[pl.ds(off, C)], rows_v)
        pltpu.sync_copy(rows_v, acc_sh.at[idx_v], add=True)

    plsc.subcore_barrier()
    pltpu.sync_copy(
        acc_sh.at[pl.ds(s * ROWS_PER_TILE, ROWS_PER_TILE)],
        out_hbm.at[c].at[pl.ds(s * ROWS_PER_TILE, ROWS_PER_TILE)],
    )


def _sc_scatter(msg, dst):
    mesh = plsc.VectorSubcoreMesh(core_axis_name="c", subcore_axis_name="s")
    z = jnp.zeros((ROWS_PER_TILE, MSG), jnp.float32)
    f = pl.kernel(
        _scatter_body,
        out_type=jax.ShapeDtypeStruct((NC, NAGG, MSG), jnp.float32),
        mesh=mesh,
        scratch_types=[
            pltpu.VMEM((C,), jnp.int32),
            pltpu.VMEM((C, MSG), jnp.float32),
            pltpu.VMEM_SHARED((NAGG, MSG), jnp.float32),
        ],
    )
    return f(msg, dst, z)


def _silu(v):
    return v * jax.nn.sigmoid(v)


def _emlp_body(xi_ref, xj_ref, ea_ref, w1a, w1b, w1c, b1, w2, b2, w3, b3,
               out_ref):
    h = (jnp.dot(xi_ref[...], w1a[...], preferred_element_type=jnp.float32)
         + jnp.dot(xj_ref[...], w1b[...], preferred_element_type=jnp.float32)
         + jnp.dot(ea_ref[...], w1c[...], preferred_element_type=jnp.float32)
         + b1[...])
    h = _silu(h)
    h = _silu(jnp.dot(h, w2[...], preferred_element_type=jnp.float32) + b2[...])
    out_ref[...] = (
        jnp.dot(h, w3[...], preferred_element_type=jnp.float32) + b3[...])


def _tc_edge_mlp(xi, xj, ea, mW1, mb1, mW2, mb2, mW3, mb3):
    BE = 1280
    grid = (E // BE,)
    w1a = mW1[:NF]
    w1b = mW1[NF:2 * NF]
    w1c = mW1[2 * NF:]
    full = lambda shape: pl.BlockSpec(shape, lambda i: (0,) * len(shape))
    return pl.pallas_call(
        _emlp_body,
        grid=grid,
        in_specs=[
            pl.BlockSpec((BE, NF), lambda i: (i, 0)),
            pl.BlockSpec((BE, NF), lambda i: (i, 0)),
            pl.BlockSpec((BE, NEF), lambda i: (i, 0)),
            full((NF, HID)),
            full((NF, HID)),
            full((NEF, HID)),
            full((1, HID)),
            full((HID, HID)),
            full((1, HID)),
            full((HID, MSG)),
            full((1, MSG)),
        ],
        out_specs=pl.BlockSpec((BE, MSG), lambda i: (i, 0)),
        out_shape=jax.ShapeDtypeStruct((E, MSG), jnp.float32),
    )(xi, xj, ea, w1a, w1b, w1c, mb1.reshape(1, HID), mW2,
      mb2.reshape(1, HID), mW3, mb3.reshape(1, MSG))


def _nmlp_body(aggr2_ref, x_ref, batch_ref, w1a, w1b, b1, w2, b2, w3, b3,
               lw, lb, out_ref, pool_acc, cnt_acc):
    i = pl.program_id(0)
    nb = pl.num_programs(0)

    @pl.when(i == 0)
    def _():
        pool_acc[...] = jnp.zeros_like(pool_acc)
        cnt_acc[...] = jnp.zeros_like(cnt_acc)

    aggr = aggr2_ref[0] + aggr2_ref[1]
    h = (jnp.dot(x_ref[...], w1a[...], preferred_element_type=jnp.float32)
         + jnp.dot(aggr, w1b[...], preferred_element_type=jnp.float32)
         + b1[...])
    h = _silu(h)
    h = _silu(jnp.dot(h, w2[...], preferred_element_type=jnp.float32) + b2[...])
    h = jnp.dot(h, w3[...], preferred_element_type=jnp.float32) + b3[...]

    ids = batch_ref[...].reshape(1, -1)
    iota = lax.broadcasted_iota(jnp.int32, (NG, ids.shape[1]), 0)
    onehot = (iota == ids).astype(jnp.float32)
    pool_acc[...] += jnp.dot(onehot, h, preferred_element_type=jnp.float32)
    cnt = jnp.sum(onehot, axis=1, keepdims=True)
    cnt_acc[...] += jnp.broadcast_to(cnt, cnt_acc.shape)

    @pl.when(i == nb - 1)
    def _():
        pooled = pool_acc[...] / jnp.maximum(cnt_acc[...], 1.0)
        out_ref[...] = (
            jnp.dot(pooled, lw[...], preferred_element_type=jnp.float32)
            + lb[...])


def _tc_node_mlp(aggr2, x, batch, nW1, nb1, nW2, nb2, nW3, nb3, lW, lb):
    BN = 400
    nblocks = N // BN
    batch3 = batch.reshape(nblocks, 1, BN)
    w1a = nW1[:NF]
    w1b = nW1[NF:]
    full = lambda shape: pl.BlockSpec(shape, lambda i: (0,) * len(shape))
    return pl.pallas_call(
        _nmlp_body,
        grid=(nblocks,),
        in_specs=[
            pl.BlockSpec((NC, BN, MSG), lambda i: (0, i, 0)),
            pl.BlockSpec((BN, NF), lambda i: (i, 0)),
            pl.BlockSpec((1, 1, BN), lambda i: (i, 0, 0)),
            full((NF, HID)),
            full((MSG, HID)),
            full((1, HID)),
            full((HID, HID)),
            full((1, HID)),
            full((HID, NH)),
            full((1, NH)),
            full((NH, NP)),
            full((1, NP)),
        ],
        out_specs=pl.BlockSpec((NG, NP), lambda i: (0, 0)),
        out_shape=jax.ShapeDtypeStruct((NG, NP), jnp.float32),
        scratch_shapes=[
            pltpu.VMEM((NG, NH), jnp.float32),
            pltpu.VMEM((NG, NH), jnp.float32),
        ],
    )(aggr2, x, batch3, w1a, w1b, nb1.reshape(1, HID), nW2,
      nb2.reshape(1, HID), nW3, nb3.reshape(1, NH), lW, lb.reshape(1, NP))


def kernel(x, edge_index, edge_attr, batch,
           mW1, mb1, mW2, mb2, mW3, mb3,
           nW1, nb1, nW2, nb2, nW3, nb3,
           lW, lb):
    src = edge_index[0]
    dst = edge_index[1]
    xj, xi = _sc_gather(x, src, dst)
    msg = _tc_edge_mlp(xi, xj, edge_attr, mW1, mb1, mW2, mb2, mW3, mb3)
    aggr2 = _sc_scatter(msg, dst)
    return _tc_node_mlp(aggr2, x, batch, nW1, nb1, nW2, nb2, nW3, nb3, lW, lb)


# speedup vs baseline: 2.7974x; 1.0083x over previous
import functools

import jax
import jax.numpy as jnp
from jax import lax
from jax.experimental import pallas as pl
from jax.experimental.pallas import tpu as pltpu
from jax.experimental.pallas import tpu_sc as plsc

N = 10000
E = 320000
NF = 128
NEF = 16
MSG = 128
HID = 300
NH = 128
NP = 2
NG = 64

NC = 2
NS = 16
NW = NC * NS
PER_W = E // NW
C = 80
NCHUNK = PER_W // C
NAGG = 10240
ROWS_PER_TILE = NAGG // NS


def _gather_body(x_hbm, src_hbm, dst_hbm, xj_hbm, xi_hbm,
                 idx_a, rows_a, idx_b, rows_b, sem_a, sem_b):
    c = lax.axis_index("c")
    s = lax.axis_index("s")
    base = (c * NS + s) * PER_W

    @pl.loop(0, NCHUNK)
    def _(j):
        off = base + j * C
        pltpu.sync_copy(src_hbm.at[pl.ds(off, C)], idx_a)
        cp_a = pltpu.async_copy(x_hbm.at[idx_a], rows_a, sem_a)
        pltpu.sync_copy(dst_hbm.at[pl.ds(off, C)], idx_b)
        cp_b = pltpu.async_copy(x_hbm.at[idx_b], rows_b, sem_b)
        cp_a.wait()
        pltpu.sync_copy(rows_a, xj_hbm.at[pl.ds(off, C)])
        cp_b.wait()
        pltpu.sync_copy(rows_b, xi_hbm.at[pl.ds(off, C)])


def _sc_gather(x, src, dst):
    mesh = plsc.VectorSubcoreMesh(core_axis_name="c", subcore_axis_name="s")
    f = pl.kernel(
        _gather_body,
        out_type=(
            jax.ShapeDtypeStruct((E, NF), jnp.float32),
            jax.ShapeDtypeStruct((E, NF), jnp.float32),
        ),
        mesh=mesh,
        scratch_types=[
            pltpu.VMEM((C,), jnp.int32),
            pltpu.VMEM((C, NF), jnp.float32),
            pltpu.VMEM((C,), jnp.int32),
            pltpu.VMEM((C, NF), jnp.float32),
            pltpu.SemaphoreType.DMA,
            pltpu.SemaphoreType.DMA,
        ],
    )
    return f(x, src, dst)


def _scatter_body(msg_hbm, dst_hbm, z_hbm, out_hbm, idx_v, rows_v, acc_sh):
    c = lax.axis_index("c")
    s = lax.axis_index("s")
    pltpu.sync_copy(z_hbm, acc_sh.at[pl.ds(s * ROWS_PER_TILE, ROWS_PER_TILE)])
    plsc.subcore_barrier()

    base = (c * NS + s) * PER_W

    @pl.loop(0, NCHUNK)
    def _(j):
        off = base + j * C
        pltpu.sync_copy(dst_hbm.at[pl.ds(off, C)], idx_v)
        pltpu.sync_copy(msg_hbm.at[pl.ds(off, C)], rows_v)
        pltpu.sync_copy(rows_v, acc_sh.at[idx_v], add=True)

    plsc.subcore_barrier()
    pltpu.sync_copy(
        acc_sh.at[pl.ds(s * ROWS_PER_TILE, ROWS_PER_TILE)],
        out_hbm.at[c].at[pl.ds(s * ROWS_PER_TILE, ROWS_PER_TILE)],
    )


def _sc_scatter(msg, dst):
    mesh = plsc.VectorSubcoreMesh(core_axis_name="c", subcore_axis_name="s")
    z = jnp.zeros((ROWS_PER_TILE, MSG), jnp.float32)
    f = pl.kernel(
        _scatter_body,
        out_type=jax.ShapeDtypeStruct((NC, NAGG, MSG), jnp.float32),
        mesh=mesh,
        scratch_types=[
            pltpu.VMEM((C,), jnp.int32),
            pltpu.VMEM((C, MSG), jnp.float32),
            pltpu.VMEM_SHARED((NAGG, MSG), jnp.float32),
        ],
    )
    return f(msg, dst, z)


def _silu(v):
    return v * jax.nn.sigmoid(v)


def _bdot(a, b):
    return jnp.dot(a.astype(jnp.bfloat16), b.astype(jnp.bfloat16),
                   preferred_element_type=jnp.float32)


def _emlp_body(xi_ref, xj_ref, ea_ref, w1a, w1b, w1c, b1, w2, b2, w3, b3,
               out_ref):
    h = (_bdot(xi_ref[...], w1a[...])
         + _bdot(xj_ref[...], w1b[...])
         + _bdot(ea_ref[...], w1c[...])
         + b1[...])
    h = _silu(h)
    h = _silu(_bdot(h, w2[...]) + b2[...])
    out_ref[...] = _bdot(h, w3[...]) + b3[...]


def _tc_edge_mlp(xi, xj, ea, mW1, mb1, mW2, mb2, mW3, mb3):
    BE = 1280
    grid = (E // BE,)
    w1a = mW1[:NF]
    w1b = mW1[NF:2 * NF]
    w1c = mW1[2 * NF:]
    full = lambda shape: pl.BlockSpec(shape, lambda i: (0,) * len(shape))
    return pl.pallas_call(
        _emlp_body,
        grid=grid,
        in_specs=[
            pl.BlockSpec((BE, NF), lambda i: (i, 0)),
            pl.BlockSpec((BE, NF), lambda i: (i, 0)),
            pl.BlockSpec((BE, NEF), lambda i: (i, 0)),
            full((NF, HID)),
            full((NF, HID)),
            full((NEF, HID)),
            full((1, HID)),
            full((HID, HID)),
            full((1, HID)),
            full((HID, MSG)),
            full((1, MSG)),
        ],
        out_specs=pl.BlockSpec((BE, MSG), lambda i: (i, 0)),
        out_shape=jax.ShapeDtypeStruct((E, MSG), jnp.float32),
    )(xi, xj, ea, w1a, w1b, w1c, mb1.reshape(1, HID), mW2,
      mb2.reshape(1, HID), mW3, mb3.reshape(1, MSG))


def _nmlp_body(aggr2_ref, x_ref, batch_ref, w1a, w1b, b1, w2, b2, w3, b3,
               lw, lb, out_ref, pool_acc, cnt_acc):
    i = pl.program_id(0)
    nb = pl.num_programs(0)

    @pl.when(i == 0)
    def _():
        pool_acc[...] = jnp.zeros_like(pool_acc)
        cnt_acc[...] = jnp.zeros_like(cnt_acc)

    aggr = aggr2_ref[0] + aggr2_ref[1]
    h = (_bdot(x_ref[...], w1a[...])
         + _bdot(aggr, w1b[...])
         + b1[...])
    h = _silu(h)
    h = _silu(_bdot(h, w2[...]) + b2[...])
    h = _bdot(h, w3[...]) + b3[...]

    ids = batch_ref[...].reshape(1, -1)
    iota = lax.broadcasted_iota(jnp.int32, (NG, ids.shape[1]), 0)
    onehot = (iota == ids).astype(jnp.float32)
    pool_acc[...] += jnp.dot(onehot, h, preferred_element_type=jnp.float32)
    cnt = jnp.sum(onehot, axis=1, keepdims=True)
    cnt_acc[...] += jnp.broadcast_to(cnt, cnt_acc.shape)

    @pl.when(i == nb - 1)
    def _():
        pooled = pool_acc[...] / jnp.maximum(cnt_acc[...], 1.0)
        out_ref[...] = (
            jnp.dot(pooled, lw[...], preferred_element_type=jnp.float32)
            + lb[...])


def _tc_node_mlp(aggr2, x, batch, nW1, nb1, nW2, nb2, nW3, nb3, lW, lb):
    BN = 400
    nblocks = N // BN
    batch3 = batch.reshape(nblocks, 1, BN)
    w1a = nW1[:NF]
    w1b = nW1[NF:]
    full = lambda shape: pl.BlockSpec(shape, lambda i: (0,) * len(shape))
    return pl.pallas_call(
        _nmlp_body,
        grid=(nblocks,),
        in_specs=[
            pl.BlockSpec((NC, BN, MSG), lambda i: (0, i, 0)),
            pl.BlockSpec((BN, NF), lambda i: (i, 0)),
            pl.BlockSpec((1, 1, BN), lambda i: (i, 0, 0)),
            full((NF, HID)),
            full((MSG, HID)),
            full((1, HID)),
            full((HID, HID)),
            full((1, HID)),
            full((HID, NH)),
            full((1, NH)),
            full((NH, NP)),
            full((1, NP)),
        ],
        out_specs=pl.BlockSpec((NG, NP), lambda i: (0, 0)),
        out_shape=jax.ShapeDtypeStruct((NG, NP), jnp.float32),
        scratch_shapes=[
            pltpu.VMEM((NG, NH), jnp.float32),
            pltpu.VMEM((NG, NH), jnp.float32),
        ],
    )(aggr2, x, batch3, w1a, w1b, nb1.reshape(1, HID), nW2,
      nb2.reshape(1, HID), nW3, nb3.reshape(1, NH), lW, lb.reshape(1, NP))


def kernel(x, edge_index, edge_attr, batch,
           mW1, mb1, mW2, mb2, mW3, mb3,
           nW1, nb1, nW2, nb2, nW3, nb3,
           lW, lb):
    src = edge_index[0]
    dst = edge_index[1]
    xj, xi = _sc_gather(x, src, dst)
    msg = _tc_edge_mlp(xi, xj, edge_attr, mW1, mb1, mW2, mb2, mW3, mb3)
    aggr2 = _sc_scatter(msg, dst)
    return _tc_node_mlp(aggr2, x, batch, nW1, nb1, nW2, nb2, nW3, nb3, lW, lb)
